# baseline (speedup 1.0000x reference)
import functools

import jax
import jax.numpy as jnp
from jax import lax
from jax.experimental import pallas as pl
from jax.experimental.pallas import tpu as pltpu
from jax.experimental.pallas import tpu_sc as plsc

N_FACTORS = 64
BATCH = 16384
NC = 2
NS = 16
LANES = 16
NW = NC * NS
B_PER_W = BATCH // NW
GCHUNK = 128
NG = B_PER_W // GCHUNK
CHUNKS = N_FACTORS // LANES


def _bpr_body(user_idx, item_i_idx, item_j_idx, uw, iw, out,
              idx_u, idx_i, idx_j, u_rows, vi_rows, vj_rows, out_v, sem):
    wid = lax.axis_index("s") * NC + lax.axis_index("c")
    row0 = wid * NG

    pltpu.sync_copy(user_idx.at[pl.ds(row0, NG)], idx_u)
    pltpu.sync_copy(item_i_idx.at[pl.ds(row0, NG)], idx_i)
    pltpu.sync_copy(item_j_idx.at[pl.ds(row0, NG)], idx_j)

    copies = []
    for j in range(NG):
        dst = pl.ds(j * GCHUNK, GCHUNK)
        copies.append(pltpu.async_copy(uw.at[idx_u.at[j]], u_rows.at[dst], sem))
        copies.append(pltpu.async_copy(iw.at[idx_i.at[j]], vi_rows.at[dst], sem))
        copies.append(pltpu.async_copy(iw.at[idx_j.at[j]], vj_rows.at[dst], sem))
    for c in copies:
        c.wait()

    lane = lax.iota(jnp.int32, LANES)

    def group_body(g, carry):
        base = g * LANES
        res = jnp.zeros((LANES,), jnp.float32)
        for i in range(LANES):
            r = base + i
            acc = jnp.zeros((LANES,), jnp.float32)
            for c in range(CHUNKS):
                sl = pl.ds(c * LANES, LANES)
                acc = acc + u_rows[r, sl] * (vi_rows[r, sl] - vj_rows[r, sl])
            res = jnp.where(lane == i, jnp.sum(acc), res)
        out_v[pl.ds(base, LANES)] = res
        return carry

    lax.fori_loop(0, B_PER_W // LANES, group_body, 0)

    pltpu.sync_copy(out_v, out.at[pl.ds(wid * B_PER_W, B_PER_W)])


@functools.partial(
    pl.kernel,
    mesh=plsc.VectorSubcoreMesh(core_axis_name="c", subcore_axis_name="s"),
    out_type=jax.ShapeDtypeStruct((BATCH,), jnp.float32),
    compiler_params=pltpu.CompilerParams(
        needs_layout_passes=False, use_tc_tiling_on_sc=False),
    scratch_types=[
        pltpu.VMEM((NG, GCHUNK), jnp.int32),
        pltpu.VMEM((NG, GCHUNK), jnp.int32),
        pltpu.VMEM((NG, GCHUNK), jnp.int32),
        pltpu.VMEM((B_PER_W, N_FACTORS), jnp.float32),
        pltpu.VMEM((B_PER_W, N_FACTORS), jnp.float32),
        pltpu.VMEM((B_PER_W, N_FACTORS), jnp.float32),
        pltpu.VMEM((B_PER_W,), jnp.float32),
        pltpu.SemaphoreType.DMA,
    ],
)
def _bpr(user_idx, item_i_idx, item_j_idx, uw, iw, out, *scratch):
    _bpr_body(user_idx, item_i_idx, item_j_idx, uw, iw, out, *scratch)


def kernel(user, item_i, item_j, embed_user_w, embed_item_w):
    user = user.astype(jnp.int32).reshape(BATCH // GCHUNK, GCHUNK)
    item_i = item_i.astype(jnp.int32).reshape(BATCH // GCHUNK, GCHUNK)
    item_j = item_j.astype(jnp.int32).reshape(BATCH // GCHUNK, GCHUNK)
    return _bpr(user, item_i, item_j, embed_user_w, embed_item_w)
